# Initial kernel scaffold; baseline (speedup 1.0000x reference)
import jax
import jax.numpy as jnp
from jax.experimental import pallas as pl


def kernel(xyz, points, query_pts):
    raise NotImplementedError("write your pallas kernel here")


# speedup vs baseline: 2.2979x; 2.2979x over previous
import functools

import jax
import jax.numpy as jnp
from jax import lax
from jax.experimental import pallas as pl

_B, _C, _N, _D = 64, 3, 16384, 64
_K = 64
_R2 = 0.2 * 0.2
_S = 128
_NEG = -3.0e38


def _ball_max_kernel(d2_ref, xyz_ref, pts_ref, out_ref):
    x = xyz_ref[0]
    d2 = d2_ref[0]
    m = jnp.logical_not(d2 > _R2)
    mf = jnp.where(m, 1.0, 0.0).astype(jnp.float32)

    iota = lax.broadcasted_iota(jnp.int32, (_S, _S), 0)
    iota_t = lax.broadcasted_iota(jnp.int32, (_S, _S), 1)
    incl = jnp.where(iota <= iota_t, 1.0, 0.0).astype(jnp.float32)
    strict = jnp.where(iota_t < iota, 1.0, 0.0).astype(jnp.float32)
    inrow = lax.dot_general(mf, incl, (((1,), (0,)), ((), ())),
                            precision=lax.Precision.HIGHEST)
    rowsum = inrow[:, _S - 1:_S]
    rowpre = lax.dot_general(strict, rowsum, (((1,), (0,)), ((), ())),
                             precision=lax.Precision.HIGHEST)
    rank = inrow + rowpre
    sel = jnp.logical_and(m, rank <= float(_K))

    total = rowpre[_S - 1, 0] + rowsum[_S - 1, 0]
    last = jnp.logical_and(iota == _S - 1, iota_t == _S - 1)
    sel = jnp.logical_or(sel, jnp.logical_and(total == 0.0, last))

    pts = pts_ref[0]
    ftmax = jnp.max(jnp.where(sel[None], pts, _NEG), axis=(1, 2))
    xyzmax = jnp.max(jnp.where(sel[None], x, _NEG), axis=(1, 2))
    row = jnp.concatenate(
        [xyzmax, ftmax, jnp.zeros((_S - _C - _D,), jnp.float32)])
    out_ref[0, 0] = row


def _ball_max(d2, xyz, points):
    d2r = d2.reshape(_B, _S, _S)
    xyzr = xyz.reshape(_B, _C, _S, _S)
    ptsr = points.reshape(_B, _D, _S, _S)
    out = pl.pallas_call(
        _ball_max_kernel,
        grid=(_B,),
        in_specs=[
            pl.BlockSpec((1, _S, _S), lambda b: (b, 0, 0)),
            pl.BlockSpec((1, _C, _S, _S), lambda b: (b, 0, 0, 0)),
            pl.BlockSpec((1, _D, _S, _S), lambda b: (b, 0, 0, 0)),
        ],
        out_specs=pl.BlockSpec((1, 1, _S), lambda b: (b, 0, 0)),
        out_shape=jax.ShapeDtypeStruct((_B, 1, _S), jnp.float32),
    )(d2r, xyzr, ptsr)
    return out


def kernel(xyz, points, query_pts):
    xyz_t = jnp.transpose(xyz, (0, 2, 1))
    new_q = query_pts.reshape(-1, 1, 3)
    sqrdists = -2.0 * jnp.matmul(new_q, jnp.transpose(xyz_t, (0, 2, 1)))
    sqrdists = sqrdists + jnp.sum(new_q ** 2, axis=-1)[..., None]
    sqrdists = sqrdists + jnp.sum(xyz_t ** 2, axis=-1)[:, None, :]
    out = _ball_max(sqrdists[:, 0, :], xyz, points)
    new_xyz = query_pts.reshape(-1, 1, 3)
    new_points = out[:, 0, :_C + _D]
    return (new_xyz, new_points)
